# baseline (speedup 1.0000x reference)
import functools

import jax
import jax.numpy as jnp
from jax import lax
from jax.experimental import pallas as pl
from jax.experimental.pallas import tpu as pltpu
from jax.experimental.pallas import tpu_sc as plsc

NC = 2
NS = 16
NW = NC * NS

CHUNK = 128
GROUP = 13


def _make_gather(total, v, d):
    per_w = total // NW
    n_chunk = per_w // CHUNK
    n_group = n_chunk // GROUP
    rows = GROUP * CHUNK
    assert per_w * NW == total and n_chunk * CHUNK == per_w
    assert n_group * GROUP == n_chunk and n_group % 2 == 0

    mesh = plsc.VectorSubcoreMesh(core_axis_name="c", subcore_axis_name="s")

    @functools.partial(
        pl.kernel,
        mesh=mesh,
        compiler_params=pltpu.CompilerParams(use_tc_tiling_on_sc=False),
        out_type=jax.ShapeDtypeStruct((total, d), jnp.bfloat16),
        scratch_types=[
            pltpu.VMEM((n_chunk, CHUNK), jnp.int32),
            pltpu.VMEM((2, rows, d), jnp.bfloat16),
            pltpu.SemaphoreType.DMA,
            pltpu.SemaphoreType.DMA,
        ],
    )
    def gather_kernel(idx_hbm, tab_hbm, out_hbm, idx_v, rows_v, gsem, osem):
        wid = lax.axis_index("s") * NC + lax.axis_index("c")
        pltpu.sync_copy(idx_hbm.at[pl.ds(wid * n_chunk, n_chunk)], idx_v)

        def fire(g, buf):
            return [pltpu.async_copy(
                        tab_hbm.at[idx_v.at[g * GROUP + b]],
                        rows_v.at[buf, pl.ds(b * CHUNK, CHUNK)],
                        gsem)
                    for b in range(GROUP)]

        def write_out(g, buf):
            return pltpu.async_copy(
                rows_v.at[buf],
                out_hbm.at[pl.ds(wid * per_w + g * rows, rows)],
                osem)

        def pair_body(p, carry):
            g0 = p * 2
            h0 = fire(g0, 0)
            for h in h0:
                h.wait()
            w0 = write_out(g0, 0)
            h1 = fire(g0 + 1, 1)
            for h in h1:
                h.wait()
            w1 = write_out(g0 + 1, 1)
            w0.wait()
            w1.wait()
            return carry

        lax.fori_loop(0, n_group // 2, pair_body, 0)

    return gather_kernel


def kernel(x, table):
    b, f = x.shape
    v, d = table.shape
    total = b * f
    idx = x.reshape(total // CHUNK, CHUNK).astype(jnp.int32)
    out = _make_gather(total, v, d)(idx, table.astype(jnp.bfloat16))
    return out.astype(jnp.float32).reshape(b, f * d)


# speedup vs baseline: 1.5467x; 1.5467x over previous
import functools

import jax
import jax.numpy as jnp
from jax import lax
from jax.experimental import pallas as pl
from jax.experimental.pallas import tpu as pltpu
from jax.experimental.pallas import tpu_sc as plsc

NC = 2
NS = 16
NW = NC * NS

CHUNK = 128
GROUP = 13


def _make_gather(total, v, d):
    per_w = total // NW
    n_chunk = per_w // CHUNK
    n_group = n_chunk // GROUP
    rows = GROUP * CHUNK
    assert per_w * NW == total and n_chunk * CHUNK == per_w
    assert n_group * GROUP == n_chunk and n_group % 2 == 0

    mesh = plsc.VectorSubcoreMesh(core_axis_name="c", subcore_axis_name="s")

    @functools.partial(
        pl.kernel,
        mesh=mesh,
        compiler_params=pltpu.CompilerParams(use_tc_tiling_on_sc=False),
        out_type=jax.ShapeDtypeStruct((total, d), jnp.float32),
        scratch_types=[
            pltpu.VMEM((n_chunk, CHUNK), jnp.int32),
            pltpu.VMEM((2, rows, d), jnp.float32),
            pltpu.SemaphoreType.DMA,
            pltpu.SemaphoreType.DMA,
        ],
    )
    def gather_kernel(idx_hbm, tab_hbm, out_hbm, idx_v, rows_v, gsem, osem):
        wid = lax.axis_index("s") * NC + lax.axis_index("c")
        pltpu.sync_copy(idx_hbm.at[pl.ds(wid * n_chunk, n_chunk)], idx_v)

        def fire(g, buf):
            return [pltpu.async_copy(
                        tab_hbm.at[idx_v.at[g * GROUP + b]],
                        rows_v.at[buf, pl.ds(b * CHUNK, CHUNK)],
                        gsem)
                    for b in range(GROUP)]

        def write_out(g, buf):
            return pltpu.async_copy(
                rows_v.at[buf],
                out_hbm.at[pl.ds(wid * per_w + g * rows, rows)],
                osem)

        def pair_body(p, carry):
            g0 = p * 2
            h0 = fire(g0, 0)
            for h in h0:
                h.wait()
            w0 = write_out(g0, 0)
            h1 = fire(g0 + 1, 1)
            for h in h1:
                h.wait()
            w1 = write_out(g0 + 1, 1)
            w0.wait()
            w1.wait()
            return carry

        lax.fori_loop(0, n_group // 2, pair_body, 0)

    return gather_kernel


def kernel(x, table):
    b, f = x.shape
    v, d = table.shape
    total = b * f
    idx = x.reshape(total // CHUNK, CHUNK).astype(jnp.int32)
    out = _make_gather(total, v, d)(idx, table)
    return out.reshape(b, f * d)
